# Initial kernel scaffold; baseline (speedup 1.0000x reference)
import jax
import jax.numpy as jnp
from jax.experimental import pallas as pl


def kernel(x, table, W, b):
    raise NotImplementedError("write your pallas kernel here")


# speedup vs baseline: 1.8299x; 1.8299x over previous
import functools

import jax
import jax.numpy as jnp
from jax import lax
from jax.experimental import pallas as pl
from jax.experimental.pallas import tpu as pltpu
from jax.experimental.pallas import tpu_sc as plsc


def _build_sc_logits(B, T, E, NC, NS, R):
    NW = NC * NS
    RPW = B // NW
    NG = RPW // R
    TPG = R * T
    D = T * E
    NCH = D // 16
    EC = E // 16

    blocks = []
    off = 0
    while off < TPG:
        n = min(128, TPG - off)
        blocks.append((off, n))
        off += n

    mesh = plsc.VectorSubcoreMesh(core_axis_name="c", subcore_axis_name="s",
                                  num_cores=NC, num_subcores=NS)

    @functools.partial(
        pl.kernel,
        out_type=(jax.ShapeDtypeStruct((B,), jnp.float32),
                  jax.ShapeDtypeStruct((B,), jnp.float32)),
        mesh=mesh,
        compiler_params=pltpu.CompilerParams(needs_layout_passes=False,
                                             use_tc_tiling_on_sc=False),
        scratch_types=[
            pltpu.VMEM((D,), jnp.float32),
            pltpu.VMEM((D,), jnp.float32),
            pltpu.VMEM((TPG,), jnp.int32),
            pltpu.VMEM((TPG, E), jnp.float32),
            pltpu.VMEM((RPW * 16,), jnp.float32),
            pltpu.VMEM((RPW * 16,), jnp.float32),
            pltpu.VMEM((RPW,), jnp.float32),
            pltpu.VMEM((RPW,), jnp.float32),
            pltpu.SemaphoreType.DMA,
        ],
    )
    def sc_logits(x_hbm, tab_hbm, w0_hbm, w1_hbm, l0_hbm, l1_hbm,
                  w0v, w1v, idxv, rows, p0, p1, o0v, o1v, sem):
        wid = lax.axis_index("s") * NC + lax.axis_index("c")
        tok_base = wid * (RPW * T)
        pltpu.sync_copy(w0_hbm, w0v)
        pltpu.sync_copy(w1_hbm, w1v)

        @pl.loop(0, NG)
        def _group(g):
            pltpu.sync_copy(x_hbm.at[pl.ds(tok_base + g * TPG, TPG)], idxv)
            cps = []
            for boff, bn in blocks:
                cps.append(pltpu.async_copy(
                    tab_hbm.at[idxv.at[pl.ds(boff, bn)]],
                    rows.at[pl.ds(boff, bn)], sem))
            for cp in cps:
                cp.wait()

            zero = jnp.zeros((16,), jnp.float32)
            init = (zero,) * (2 * R)

            def body(i, accs):
                off = i * 16
                t = i // EC
                e0 = (i % EC) * 16
                w0c = w0v[pl.ds(off, 16)]
                w1c = w1v[pl.ds(off, 16)]
                out = []
                for r in range(R):
                    v = rows[r * T + t, pl.ds(e0, 16)]
                    out.append(accs[2 * r] + v * w0c)
                    out.append(accs[2 * r + 1] + v * w1c)
                return tuple(out)

            accs = lax.fori_loop(0, NCH, body, init)
            for r in range(R):
                p0[pl.ds((g * R + r) * 16, 16)] = accs[2 * r]
                p1[pl.ds((g * R + r) * 16, 16)] = accs[2 * r + 1]

        lanes16 = lax.iota(jnp.int32, 16) * 16

        @pl.loop(0, RPW // 16)
        def _reduce(j):
            base = j * 256 + lanes16
            s0 = jnp.zeros((16,), jnp.float32)
            s1 = jnp.zeros((16,), jnp.float32)
            for c in range(16):
                s0 = s0 + plsc.load_gather(p0, [base + c])
                s1 = s1 + plsc.load_gather(p1, [base + c])
            o0v[pl.ds(j * 16, 16)] = s0
            o1v[pl.ds(j * 16, 16)] = s1

        out_base = wid * RPW
        pltpu.sync_copy(o0v, l0_hbm.at[pl.ds(out_base, RPW)])
        pltpu.sync_copy(o1v, l1_hbm.at[pl.ds(out_base, RPW)])

    return sc_logits


def _softmax_tc(l0, l1, b):
    def body(b_ref, l0_ref, l1_ref, o0_ref, o1_ref):
        a0 = l0_ref[...] + b_ref[0]
        a1 = l1_ref[...] + b_ref[1]
        m = jnp.maximum(a0, a1)
        s = m + jnp.log(jnp.exp(a0 - m) + jnp.exp(a1 - m))
        o0_ref[...] = a0 - s
        o1_ref[...] = a1 - s

    return pl.pallas_call(
        body,
        out_shape=(jax.ShapeDtypeStruct(l0.shape, jnp.float32),
                   jax.ShapeDtypeStruct(l1.shape, jnp.float32)),
        in_specs=[
            pl.BlockSpec(memory_space=pltpu.SMEM),
            pl.BlockSpec(memory_space=pltpu.VMEM),
            pl.BlockSpec(memory_space=pltpu.VMEM),
        ],
        out_specs=(pl.BlockSpec(memory_space=pltpu.VMEM),
                   pl.BlockSpec(memory_space=pltpu.VMEM)),
    )(b, l0, l1)


def kernel(x, table, W, b):
    B, T = x.shape
    _, E = table.shape
    x_flat = x.reshape(-1).astype(jnp.int32)
    w0 = W[:, 0].astype(jnp.float32)
    w1 = W[:, 1].astype(jnp.float32)
    sc = _build_sc_logits(B, T, E, 2, 16, 4)
    l0, l1 = sc(x_flat, table, w0, w1)
    o0, o1 = _softmax_tc(l0.reshape(128, -1), l1.reshape(128, -1), b)
    return jnp.stack([o0.reshape(B), o1.reshape(B)], axis=-1)


# speedup vs baseline: 2.4636x; 1.3463x over previous
import functools

import jax
import jax.numpy as jnp
from jax import lax
from jax.experimental import pallas as pl
from jax.experimental.pallas import tpu as pltpu
from jax.experimental.pallas import tpu_sc as plsc


def _build_sc_logits(B, T, E, NC, NS, R):
    NW = NC * NS
    RPW = B // NW
    NG = RPW // R
    TPG = R * T
    D = T * E
    EC = E // 16
    THA = ((T // 2 + 7) // 8) * 8
    THB = T - THA

    mesh = plsc.VectorSubcoreMesh(core_axis_name="c", subcore_axis_name="s",
                                  num_cores=NC, num_subcores=NS)

    @functools.partial(
        pl.kernel,
        out_type=(jax.ShapeDtypeStruct((B,), jnp.float32),
                  jax.ShapeDtypeStruct((B,), jnp.float32)),
        mesh=mesh,
        compiler_params=pltpu.CompilerParams(needs_layout_passes=False,
                                             use_tc_tiling_on_sc=False),
        scratch_types=[
            pltpu.VMEM((D,), jnp.float32),
            pltpu.VMEM((D,), jnp.float32),
            pltpu.VMEM((TPG,), jnp.int32),
            pltpu.VMEM((TPG,), jnp.int32),
            pltpu.VMEM((R * THA, E), jnp.float32),
            pltpu.VMEM((R * THB, E), jnp.float32),
            pltpu.VMEM((RPW * 16,), jnp.float32),
            pltpu.VMEM((RPW * 16,), jnp.float32),
            pltpu.VMEM((RPW,), jnp.float32),
            pltpu.VMEM((RPW,), jnp.float32),
            pltpu.SemaphoreType.DMA,
            pltpu.SemaphoreType.DMA,
            pltpu.SemaphoreType.DMA,
        ],
    )
    def sc_logits(x_hbm, tab_hbm, w0_hbm, w1_hbm, l0_hbm, l1_hbm,
                  w0v, w1v, idx0, idx1, bufA, bufB, p0, p1, o0v, o1v,
                  semA, semB, semI):
        wid = lax.axis_index("s") * NC + lax.axis_index("c")
        tok_base = wid * (RPW * T)
        pltpu.sync_copy(w0_hbm, w0v)
        pltpu.sync_copy(w1_hbm, w1v)

        def issue_half(idxv, buf, sem, tok_off, ntok):
            for r in range(R):
                pltpu.async_copy(
                    tab_hbm.at[idxv.at[pl.ds(r * T + tok_off, ntok)]],
                    buf.at[pl.ds(r * ntok, ntok)], sem)

        def wait_half(buf, sem, ntok):
            for r in range(R):
                pltpu.make_async_copy(
                    tab_hbm.at[idx0.at[pl.ds(0, ntok)]],
                    buf.at[pl.ds(r * ntok, ntok)], sem).wait()

        def compute_half(buf, tok_off, ntok, accs):
            def body(i, accs):
                t = i // EC
                e0 = (i % EC) * 16
                w0c = w0v[pl.ds(tok_off * E + i * 16, 16)]
                w1c = w1v[pl.ds(tok_off * E + i * 16, 16)]
                out = []
                for r in range(R):
                    v = buf[r * ntok + t, pl.ds(e0, 16)]
                    out.append(accs[2 * r] + v * w0c)
                    out.append(accs[2 * r + 1] + v * w1c)
                return tuple(out)
            return lax.fori_loop(0, ntok * EC, body, accs, unroll=2)

        pltpu.sync_copy(x_hbm.at[pl.ds(tok_base, TPG)], idx0)
        issue_half(idx0, bufA, semA, 0, THA)
        issue_half(idx0, bufB, semB, THA, THB)
        pltpu.async_copy(x_hbm.at[pl.ds(tok_base + TPG, TPG)], idx1, semI)

        zero = jnp.zeros((16,), jnp.float32)

        @pl.loop(0, NG, step=2)
        def _groups(G):
            for p, (cur, nxt) in ((0, (idx0, idx1)), (1, (idx1, idx0))):
                g = G + p
                wait_half(bufA, semA, THA)
                accs = compute_half(bufA, 0, THA, (zero,) * (2 * R))

                @pl.when(g + 1 < NG)
                def _():
                    pltpu.make_async_copy(
                        x_hbm.at[pl.ds(tok_base, TPG)], nxt, semI).wait()
                    issue_half(nxt, bufA, semA, 0, THA)

                wait_half(bufB, semB, THB)
                accs = compute_half(bufB, THA, THB, accs)

                for r in range(R):
                    p0[pl.ds((g * R + r) * 16, 16)] = accs[2 * r]
                    p1[pl.ds((g * R + r) * 16, 16)] = accs[2 * r + 1]

                @pl.when(g + 1 < NG)
                def _():
                    issue_half(nxt, bufB, semB, THA, THB)

                @pl.when(g + 2 < NG)
                def _():
                    pltpu.async_copy(
                        x_hbm.at[pl.ds(tok_base + (g + 2) * TPG, TPG)],
                        cur, semI)

        lanes16 = lax.iota(jnp.int32, 16) * 16

        @pl.loop(0, RPW // 16)
        def _reduce(j):
            base = j * 256 + lanes16
            s0 = jnp.zeros((16,), jnp.float32)
            s1 = jnp.zeros((16,), jnp.float32)
            for c in range(16):
                s0 = s0 + plsc.load_gather(p0, [base + c])
                s1 = s1 + plsc.load_gather(p1, [base + c])
            o0v[pl.ds(j * 16, 16)] = s0
            o1v[pl.ds(j * 16, 16)] = s1

        out_base = wid * RPW
        pltpu.sync_copy(o0v, l0_hbm.at[pl.ds(out_base, RPW)])
        pltpu.sync_copy(o1v, l1_hbm.at[pl.ds(out_base, RPW)])

    return sc_logits


def _softmax_tc(l0, l1, b):
    def body(b_ref, l0_ref, l1_ref, o0_ref, o1_ref):
        a0 = l0_ref[...] + b_ref[0]
        a1 = l1_ref[...] + b_ref[1]
        m = jnp.maximum(a0, a1)
        s = m + jnp.log(jnp.exp(a0 - m) + jnp.exp(a1 - m))
        o0_ref[...] = a0 - s
        o1_ref[...] = a1 - s

    return pl.pallas_call(
        body,
        out_shape=(jax.ShapeDtypeStruct(l0.shape, jnp.float32),
                   jax.ShapeDtypeStruct(l1.shape, jnp.float32)),
        in_specs=[
            pl.BlockSpec(memory_space=pltpu.SMEM),
            pl.BlockSpec(memory_space=pltpu.VMEM),
            pl.BlockSpec(memory_space=pltpu.VMEM),
        ],
        out_specs=(pl.BlockSpec(memory_space=pltpu.VMEM),
                   pl.BlockSpec(memory_space=pltpu.VMEM)),
    )(b, l0, l1)


def kernel(x, table, W, b):
    B, T = x.shape
    _, E = table.shape
    x_flat = x.reshape(-1).astype(jnp.int32)
    w0 = W[:, 0].astype(jnp.float32)
    w1 = W[:, 1].astype(jnp.float32)
    sc = _build_sc_logits(B, T, E, 2, 16, 4)
    l0, l1 = sc(x_flat, table, w0, w1)
    o0, o1 = _softmax_tc(l0.reshape(128, -1), l1.reshape(128, -1), b)
    return jnp.stack([o0.reshape(B), o1.reshape(B)], axis=-1)


# speedup vs baseline: 2.7336x; 1.1096x over previous
import functools

import jax
import jax.numpy as jnp
from jax import lax
from jax.experimental import pallas as pl
from jax.experimental.pallas import tpu as pltpu
from jax.experimental.pallas import tpu_sc as plsc


def _build_sc_logits(B, T, E, V, NC, NS, R):
    NW = NC * NS
    RPW = B // NW
    NG = RPW // R
    TPG = R * T
    D = T * E
    EC = E // 16
    THA = ((T // 2 + 7) // 8) * 8
    THB = T - THA

    mesh = plsc.VectorSubcoreMesh(core_axis_name="c", subcore_axis_name="s",
                                  num_cores=NC, num_subcores=NS)

    @functools.partial(
        pl.kernel,
        out_type=(jax.ShapeDtypeStruct((B,), jnp.float32),
                  jax.ShapeDtypeStruct((B,), jnp.float32)),
        mesh=mesh,
        compiler_params=pltpu.CompilerParams(needs_layout_passes=False,
                                             use_tc_tiling_on_sc=False),
        scratch_types=[
            pltpu.VMEM((D,), jnp.float32),
            pltpu.VMEM((D,), jnp.float32),
            pltpu.VMEM((TPG,), jnp.int32),
            pltpu.VMEM((TPG,), jnp.int32),
            pltpu.VMEM((R * THA, E), jnp.float32),
            pltpu.VMEM((R * THB, E), jnp.float32),
            pltpu.VMEM((RPW * 16,), jnp.float32),
            pltpu.VMEM((RPW * 16,), jnp.float32),
            pltpu.VMEM((RPW,), jnp.float32),
            pltpu.VMEM((RPW,), jnp.float32),
            pltpu.SemaphoreType.DMA,
            pltpu.SemaphoreType.DMA,
            pltpu.SemaphoreType.DMA,
        ],
    )
    def sc_logits(x_hbm, tab1d_hbm, w0_hbm, w1_hbm, l0_hbm, l1_hbm,
                  w0v, w1v, idx0, idx1, bufA, bufB, p0, p1, o0v, o1v,
                  semA, semB, semI):
        tab_hbm = tab1d_hbm
        wid = lax.axis_index("s") * NC + lax.axis_index("c")
        tok_base = wid * (RPW * T)
        pltpu.sync_copy(w0_hbm, w0v)
        pltpu.sync_copy(w1_hbm, w1v)

        def xform_idx(idxv):
            @pl.loop(0, TPG // 16)
            def _x(c):
                u = idxv[pl.ds(c * 16, 16)]
                h = u & 2047
                r = u & 1023
                idxv[pl.ds(c * 16, 16)] = u - h + r + r + (h >> 10)

        def issue_half(idxv, buf, sem, tok_off, ntok):
            for r in range(R):
                pltpu.async_copy(
                    tab_hbm.at[idxv.at[pl.ds(r * T + tok_off, ntok)]],
                    buf.at[pl.ds(r * ntok, ntok)], sem)

        def wait_half(buf, sem, ntok):
            for r in range(R):
                pltpu.make_async_copy(
                    tab_hbm.at[idx0.at[pl.ds(0, ntok)]],
                    buf.at[pl.ds(r * ntok, ntok)], sem).wait()

        def compute_half(buf, tok_off, ntok, accs):
            def body(i, accs):
                t = i // EC
                e0 = (i % EC) * 16
                w0c = w0v[pl.ds(tok_off * E + i * 16, 16)]
                w1c = w1v[pl.ds(tok_off * E + i * 16, 16)]
                out = []
                for r in range(R):
                    v = buf[r * ntok + t, pl.ds(e0, 16)]
                    out.append(accs[2 * r] + v * w0c)
                    out.append(accs[2 * r + 1] + v * w1c)
                return tuple(out)
            return lax.fori_loop(0, ntok * EC, body, accs, unroll=2)

        pltpu.sync_copy(x_hbm.at[pl.ds(tok_base, TPG)], idx0)
        xform_idx(idx0)
        issue_half(idx0, bufA, semA, 0, THA)
        issue_half(idx0, bufB, semB, THA, THB)
        pltpu.async_copy(x_hbm.at[pl.ds(tok_base + TPG, TPG)], idx1, semI)

        zero = jnp.zeros((16,), jnp.float32)

        @pl.loop(0, NG, step=2)
        def _groups(G):
            for p, (cur, nxt) in ((0, (idx0, idx1)), (1, (idx1, idx0))):
                g = G + p
                wait_half(bufA, semA, THA)
                accs = compute_half(bufA, 0, THA, (zero,) * (2 * R))

                @pl.when(g + 1 < NG)
                def _():
                    pltpu.make_async_copy(
                        x_hbm.at[pl.ds(tok_base, TPG)], nxt, semI).wait()
                    xform_idx(nxt)
                    issue_half(nxt, bufA, semA, 0, THA)

                wait_half(bufB, semB, THB)
                accs = compute_half(bufB, THA, THB, accs)

                for r in range(R):
                    p0[pl.ds((g * R + r) * 16, 16)] = accs[2 * r]
                    p1[pl.ds((g * R + r) * 16, 16)] = accs[2 * r + 1]

                @pl.when(g + 1 < NG)
                def _():
                    issue_half(nxt, bufB, semB, THA, THB)

                @pl.when(g + 2 < NG)
                def _():
                    pltpu.async_copy(
                        x_hbm.at[pl.ds(tok_base + (g + 2) * TPG, TPG)],
                        cur, semI)

        lanes16 = lax.iota(jnp.int32, 16) * 16

        @pl.loop(0, RPW // 16)
        def _reduce(j):
            base = j * 256 + lanes16
            s0 = jnp.zeros((16,), jnp.float32)
            s1 = jnp.zeros((16,), jnp.float32)
            for c in range(16):
                s0 = s0 + plsc.load_gather(p0, [base + c])
                s1 = s1 + plsc.load_gather(p1, [base + c])
            o0v[pl.ds(j * 16, 16)] = s0
            o1v[pl.ds(j * 16, 16)] = s1

        out_base = wid * RPW
        pltpu.sync_copy(o0v, l0_hbm.at[pl.ds(out_base, RPW)])
        pltpu.sync_copy(o1v, l1_hbm.at[pl.ds(out_base, RPW)])

    return sc_logits


def _transpose_table_tc(tabT, C=1024):
    E, V = tabT.shape
    NB = (V + 2 * C - 1) // (2 * C)
    last_blk = (V + C - 1) // C - 1

    def body(l_ref, r_ref, o_ref):
        o_ref[...] = jnp.concatenate(
            [jnp.transpose(l_ref[...]), jnp.transpose(r_ref[...])], axis=1)

    return pl.pallas_call(
        body,
        grid=(NB,),
        in_specs=[
            pl.BlockSpec((E, C), lambda g: (0, jnp.minimum(2 * g, last_blk))),
            pl.BlockSpec((E, C),
                         lambda g: (0, jnp.minimum(2 * g + 1, last_blk))),
        ],
        out_specs=pl.BlockSpec((C, 2 * E), lambda g: (g, 0)),
        out_shape=jax.ShapeDtypeStruct((NB * C, 2 * E), jnp.float32),
    )(tabT, tabT)


def _softmax_tc(l0, l1, b):
    def body(b_ref, l0_ref, l1_ref, o0_ref, o1_ref):
        a0 = l0_ref[...] + b_ref[0]
        a1 = l1_ref[...] + b_ref[1]
        m = jnp.maximum(a0, a1)
        s = m + jnp.log(jnp.exp(a0 - m) + jnp.exp(a1 - m))
        o0_ref[...] = a0 - s
        o1_ref[...] = a1 - s

    return pl.pallas_call(
        body,
        out_shape=(jax.ShapeDtypeStruct(l0.shape, jnp.float32),
                   jax.ShapeDtypeStruct(l1.shape, jnp.float32)),
        in_specs=[
            pl.BlockSpec(memory_space=pltpu.SMEM),
            pl.BlockSpec(memory_space=pltpu.VMEM),
            pl.BlockSpec(memory_space=pltpu.VMEM),
        ],
        out_specs=(pl.BlockSpec(memory_space=pltpu.VMEM),
                   pl.BlockSpec(memory_space=pltpu.VMEM)),
    )(b, l0, l1)


def kernel(x, table, W, b):
    B, T = x.shape
    _, E = table.shape
    x_flat = x.reshape(-1).astype(jnp.int32)
    w0 = W[:, 0].astype(jnp.float32)
    w1 = W[:, 1].astype(jnp.float32)
    table_rm = _transpose_table_tc(table.T)
    table_rm = table_rm.reshape(table_rm.shape[0] * 2, E)
    sc = _build_sc_logits(B, T, E, table_rm.shape[0], 2, 16, 4)
    l0, l1 = sc(x_flat, table_rm, w0, w1)
    o0, o1 = _softmax_tc(l0.reshape(128, -1), l1.reshape(128, -1), b)
    return jnp.stack([o0.reshape(B), o1.reshape(B)], axis=-1)


# speedup vs baseline: 3.1325x; 1.1460x over previous
import functools

import jax
import jax.numpy as jnp
from jax import lax
from jax.experimental import pallas as pl
from jax.experimental.pallas import tpu as pltpu
from jax.experimental.pallas import tpu_sc as plsc


def _build_sc_logits(B, T, E, V, NC, NS, R):
    NW = NC * NS
    RPW = B // NW
    NG = RPW // R
    TPG = R * T
    D = T * E
    EC = E // 16
    THA = ((T // 2 + 7) // 8) * 8
    THB = T - THA

    mesh = plsc.VectorSubcoreMesh(core_axis_name="c", subcore_axis_name="s",
                                  num_cores=NC, num_subcores=NS)

    @functools.partial(
        pl.kernel,
        out_type=(jax.ShapeDtypeStruct((B,), jnp.float32),
                  jax.ShapeDtypeStruct((B,), jnp.float32)),
        mesh=mesh,
        compiler_params=pltpu.CompilerParams(needs_layout_passes=False,
                                             use_tc_tiling_on_sc=False),
        scratch_types=[
            pltpu.VMEM((D,), jnp.float32),
            pltpu.VMEM((D,), jnp.float32),
            pltpu.VMEM((TPG,), jnp.int32),
            pltpu.VMEM((TPG,), jnp.int32),
            pltpu.VMEM((R * THA, E), jnp.float32),
            pltpu.VMEM((R * THB, E), jnp.float32),
            pltpu.VMEM((RPW * 16,), jnp.float32),
            pltpu.VMEM((RPW * 16,), jnp.float32),
            pltpu.VMEM((RPW,), jnp.float32),
            pltpu.VMEM((RPW,), jnp.float32),
            pltpu.SemaphoreType.DMA,
            pltpu.SemaphoreType.DMA,
            pltpu.SemaphoreType.DMA,
        ],
    )
    def sc_logits(x_hbm, tab1d_hbm, w0_hbm, w1_hbm, l0_hbm, l1_hbm,
                  w0v, w1v, idx0, idx1, bufA, bufB, p0, p1, o0v, o1v,
                  semA, semB, semI):
        tab_hbm = tab1d_hbm
        wid = lax.axis_index("s") * NC + lax.axis_index("c")
        tok_base = wid * (RPW * T)
        pltpu.sync_copy(w0_hbm, w0v)
        pltpu.sync_copy(w1_hbm, w1v)

        def xform_idx(idxv):
            @pl.loop(0, TPG // 16)
            def _x(c):
                u = idxv[pl.ds(c * 16, 16)]
                h = u & 4095
                r = u & 2047
                idxv[pl.ds(c * 16, 16)] = u - h + r + r + (h >> 11)

        def issue_half(idxv, buf, sem, tok_off, ntok):
            for r in range(R):
                pltpu.async_copy(
                    tab_hbm.at[idxv.at[pl.ds(r * T + tok_off, ntok)]],
                    buf.at[pl.ds(r * ntok, ntok)], sem)

        def wait_half(buf, sem, ntok):
            for r in range(R):
                pltpu.make_async_copy(
                    tab_hbm.at[idx0.at[pl.ds(0, ntok)]],
                    buf.at[pl.ds(r * ntok, ntok)], sem).wait()

        def compute_half(buf, tok_off, ntok, accs):
            def body(i, accs):
                t = i // EC
                e0 = (i % EC) * 16
                w0c = w0v[pl.ds(tok_off * E + i * 16, 16)]
                w1c = w1v[pl.ds(tok_off * E + i * 16, 16)]
                out = []
                for r in range(R):
                    v = buf[r * ntok + t, pl.ds(e0, 16)]
                    out.append(accs[2 * r] + v * w0c)
                    out.append(accs[2 * r + 1] + v * w1c)
                return tuple(out)
            return lax.fori_loop(0, ntok * EC, body, accs, unroll=2)

        pltpu.sync_copy(x_hbm.at[pl.ds(tok_base, TPG)], idx0)
        xform_idx(idx0)
        issue_half(idx0, bufA, semA, 0, THA)
        issue_half(idx0, bufB, semB, THA, THB)
        pltpu.async_copy(x_hbm.at[pl.ds(tok_base + TPG, TPG)], idx1, semI)

        zero = jnp.zeros((16,), jnp.float32)

        @pl.loop(0, NG, step=2)
        def _groups(G):
            for p, (cur, nxt) in ((0, (idx0, idx1)), (1, (idx1, idx0))):
                g = G + p
                wait_half(bufA, semA, THA)
                accs = compute_half(bufA, 0, THA, (zero,) * (2 * R))

                @pl.when(g + 1 < NG)
                def _():
                    pltpu.make_async_copy(
                        x_hbm.at[pl.ds(tok_base, TPG)], nxt, semI).wait()
                    xform_idx(nxt)
                    issue_half(nxt, bufA, semA, 0, THA)

                wait_half(bufB, semB, THB)
                accs = compute_half(bufB, THA, THB, accs)

                for r in range(R):
                    p0[pl.ds((g * R + r) * 16, 16)] = accs[2 * r]
                    p1[pl.ds((g * R + r) * 16, 16)] = accs[2 * r + 1]

                @pl.when(g + 1 < NG)
                def _():
                    issue_half(nxt, bufB, semB, THA, THB)

                @pl.when(g + 2 < NG)
                def _():
                    pltpu.async_copy(
                        x_hbm.at[pl.ds(tok_base + (g + 2) * TPG, TPG)],
                        cur, semI)

        lanes16 = lax.iota(jnp.int32, 16) * 16

        @pl.loop(0, RPW // 16)
        def _reduce(j):
            base = j * 256 + lanes16
            s0 = jnp.zeros((16,), jnp.float32)
            s1 = jnp.zeros((16,), jnp.float32)
            for c in range(16):
                s0 = s0 + plsc.load_gather(p0, [base + c])
                s1 = s1 + plsc.load_gather(p1, [base + c])
            o0v[pl.ds(j * 16, 16)] = s0
            o1v[pl.ds(j * 16, 16)] = s1

        out_base = wid * RPW
        pltpu.sync_copy(o0v, l0_hbm.at[pl.ds(out_base, RPW)])
        pltpu.sync_copy(o1v, l1_hbm.at[pl.ds(out_base, RPW)])

    return sc_logits


def _transpose_table_tc(tabT, C=2048):
    E, V = tabT.shape
    NB = (V + 2 * C - 1) // (2 * C)
    last_blk = (V + C - 1) // C - 1

    def body(l_ref, r_ref, o_ref):
        o_ref[...] = jnp.concatenate(
            [jnp.transpose(l_ref[...]), jnp.transpose(r_ref[...])], axis=1)

    return pl.pallas_call(
        body,
        grid=(NB,),
        in_specs=[
            pl.BlockSpec((E, C), lambda g: (0, jnp.minimum(2 * g, last_blk))),
            pl.BlockSpec((E, C),
                         lambda g: (0, jnp.minimum(2 * g + 1, last_blk))),
        ],
        out_specs=pl.BlockSpec((C, 2 * E), lambda g: (g, 0)),
        out_shape=jax.ShapeDtypeStruct((NB * C, 2 * E), jnp.float32),
    )(tabT, tabT)


def _softmax_tc(l0, l1, b):
    def body(b_ref, l0_ref, l1_ref, o0_ref, o1_ref):
        a0 = l0_ref[...] + b_ref[0]
        a1 = l1_ref[...] + b_ref[1]
        m = jnp.maximum(a0, a1)
        s = m + jnp.log(jnp.exp(a0 - m) + jnp.exp(a1 - m))
        o0_ref[...] = a0 - s
        o1_ref[...] = a1 - s

    return pl.pallas_call(
        body,
        out_shape=(jax.ShapeDtypeStruct(l0.shape, jnp.float32),
                   jax.ShapeDtypeStruct(l1.shape, jnp.float32)),
        in_specs=[
            pl.BlockSpec(memory_space=pltpu.SMEM),
            pl.BlockSpec(memory_space=pltpu.VMEM),
            pl.BlockSpec(memory_space=pltpu.VMEM),
        ],
        out_specs=(pl.BlockSpec(memory_space=pltpu.VMEM),
                   pl.BlockSpec(memory_space=pltpu.VMEM)),
    )(b, l0, l1)


def kernel(x, table, W, b):
    B, T = x.shape
    _, E = table.shape
    x_flat = x.reshape(-1).astype(jnp.int32)
    w0 = W[:, 0].astype(jnp.float32)
    w1 = W[:, 1].astype(jnp.float32)
    table_rm = _transpose_table_tc(table.T)
    table_rm = table_rm.reshape(table_rm.shape[0] * 2, E)
    sc = _build_sc_logits(B, T, E, table_rm.shape[0], 2, 16, 4)
    l0, l1 = sc(x_flat, table_rm, w0, w1)
    o0, o1 = _softmax_tc(l0.reshape(128, -1), l1.reshape(128, -1), b)
    return jnp.stack([o0.reshape(B), o1.reshape(B)], axis=-1)


# speedup vs baseline: 3.2210x; 1.0282x over previous
import functools

import jax
import jax.numpy as jnp
from jax import lax
from jax.experimental import pallas as pl
from jax.experimental.pallas import tpu as pltpu
from jax.experimental.pallas import tpu_sc as plsc


def _build_sc_logits(B, T, E, V, NC, NS, R):
    NW = NC * NS
    RPW = B // NW
    NG = RPW // R
    TPG = R * T
    D = T * E
    EC = E // 16
    THA = ((T // 2 + 7) // 8) * 8
    THB = T - THA

    mesh = plsc.VectorSubcoreMesh(core_axis_name="c", subcore_axis_name="s",
                                  num_cores=NC, num_subcores=NS)

    @functools.partial(
        pl.kernel,
        out_type=(jax.ShapeDtypeStruct((B,), jnp.float32),
                  jax.ShapeDtypeStruct((B,), jnp.float32)),
        mesh=mesh,
        compiler_params=pltpu.CompilerParams(needs_layout_passes=False,
                                             use_tc_tiling_on_sc=False),
        scratch_types=[
            pltpu.VMEM((D,), jnp.float32),
            pltpu.VMEM((D,), jnp.float32),
            pltpu.VMEM((TPG,), jnp.int32),
            pltpu.VMEM((TPG,), jnp.int32),
            pltpu.VMEM((R * THA, E), jnp.float32),
            pltpu.VMEM((R * THB, E), jnp.float32),
            pltpu.VMEM((RPW * 16,), jnp.float32),
            pltpu.VMEM((RPW * 16,), jnp.float32),
            pltpu.VMEM((RPW,), jnp.float32),
            pltpu.VMEM((RPW,), jnp.float32),
            pltpu.SemaphoreType.DMA,
            pltpu.SemaphoreType.DMA,
            pltpu.SemaphoreType.DMA,
        ],
    )
    def sc_logits(x_hbm, tab1d_hbm, w0_hbm, w1_hbm, l0_hbm, l1_hbm,
                  w0v, w1v, idx0, idx1, bufA, bufB, p0, p1, o0v, o1v,
                  semA, semB, semI):
        tab_hbm = tab1d_hbm
        wid = lax.axis_index("s") * NC + lax.axis_index("c")
        tok_base = wid * (RPW * T)
        pltpu.sync_copy(w0_hbm, w0v)
        pltpu.sync_copy(w1_hbm, w1v)

        def issue_half(idxv, buf, sem, tok_off, ntok):
            for r in range(R):
                pltpu.async_copy(
                    tab_hbm.at[idxv.at[pl.ds(r * T + tok_off, ntok)]],
                    buf.at[pl.ds(r * ntok, ntok)], sem)

        def wait_half(buf, sem, ntok):
            for r in range(R):
                pltpu.make_async_copy(
                    tab_hbm.at[idx0.at[pl.ds(0, ntok)]],
                    buf.at[pl.ds(r * ntok, ntok)], sem).wait()

        def compute_half(buf, tok_off, ntok, accs):
            def body(i, accs):
                t = i // EC
                e0 = (i % EC) * 16
                w0c = w0v[pl.ds(tok_off * E + i * 16, 16)]
                w1c = w1v[pl.ds(tok_off * E + i * 16, 16)]
                out = []
                for r in range(R):
                    v = buf[r * ntok + t, pl.ds(e0, 16)]
                    out.append(accs[2 * r] + v * w0c)
                    out.append(accs[2 * r + 1] + v * w1c)
                return tuple(out)
            return lax.fori_loop(0, ntok * EC, body, accs, unroll=2)

        pltpu.sync_copy(x_hbm.at[pl.ds(tok_base, TPG)], idx0)
        issue_half(idx0, bufA, semA, 0, THA)
        issue_half(idx0, bufB, semB, THA, THB)
        pltpu.async_copy(x_hbm.at[pl.ds(tok_base + TPG, TPG)], idx1, semI)

        zero = jnp.zeros((16,), jnp.float32)

        @pl.loop(0, NG, step=2)
        def _groups(G):
            for p, (cur, nxt) in ((0, (idx0, idx1)), (1, (idx1, idx0))):
                g = G + p
                wait_half(bufA, semA, THA)
                accs = compute_half(bufA, 0, THA, (zero,) * (2 * R))

                @pl.when(g + 1 < NG)
                def _():
                    pltpu.make_async_copy(
                        x_hbm.at[pl.ds(tok_base, TPG)], nxt, semI).wait()
                    issue_half(nxt, bufA, semA, 0, THA)

                wait_half(bufB, semB, THB)
                accs = compute_half(bufB, THA, THB, accs)

                for r in range(R):
                    p0[pl.ds((g * R + r) * 16, 16)] = accs[2 * r]
                    p1[pl.ds((g * R + r) * 16, 16)] = accs[2 * r + 1]

                @pl.when(g + 1 < NG)
                def _():
                    issue_half(nxt, bufB, semB, THA, THB)

                @pl.when(g + 2 < NG)
                def _():
                    pltpu.async_copy(
                        x_hbm.at[pl.ds(tok_base + (g + 2) * TPG, TPG)],
                        cur, semI)

        lanes16 = lax.iota(jnp.int32, 16) * 16

        @pl.loop(0, RPW // 16)
        def _reduce(j):
            base = j * 256 + lanes16
            s0 = jnp.zeros((16,), jnp.float32)
            s1 = jnp.zeros((16,), jnp.float32)
            for c in range(16):
                s0 = s0 + plsc.load_gather(p0, [base + c])
                s1 = s1 + plsc.load_gather(p1, [base + c])
            o0v[pl.ds(j * 16, 16)] = s0
            o1v[pl.ds(j * 16, 16)] = s1

        out_base = wid * RPW
        pltpu.sync_copy(o0v, l0_hbm.at[pl.ds(out_base, RPW)])
        pltpu.sync_copy(o1v, l1_hbm.at[pl.ds(out_base, RPW)])

    return sc_logits


def _transpose_table_tc(tabT, C=2048):
    E, V = tabT.shape
    NB = (V + 2 * C - 1) // (2 * C)
    last_blk = (V + C - 1) // C - 1

    def body(l_ref, r_ref, o_ref):
        o_ref[...] = jnp.concatenate(
            [jnp.transpose(l_ref[...]), jnp.transpose(r_ref[...])], axis=1)

    return pl.pallas_call(
        body,
        grid=(NB,),
        in_specs=[
            pl.BlockSpec((E, C), lambda g: (0, jnp.minimum(2 * g, last_blk))),
            pl.BlockSpec((E, C),
                         lambda g: (0, jnp.minimum(2 * g + 1, last_blk))),
        ],
        out_specs=pl.BlockSpec((C, 2 * E), lambda g: (g, 0)),
        out_shape=jax.ShapeDtypeStruct((NB * C, 2 * E), jnp.float32),
    )(tabT, tabT)


def _softmax_tc(l0, l1, b):
    def body(b_ref, l0_ref, l1_ref, o0_ref, o1_ref):
        a0 = l0_ref[...] + b_ref[0]
        a1 = l1_ref[...] + b_ref[1]
        m = jnp.maximum(a0, a1)
        s = m + jnp.log(jnp.exp(a0 - m) + jnp.exp(a1 - m))
        o0_ref[...] = a0 - s
        o1_ref[...] = a1 - s

    return pl.pallas_call(
        body,
        out_shape=(jax.ShapeDtypeStruct(l0.shape, jnp.float32),
                   jax.ShapeDtypeStruct(l1.shape, jnp.float32)),
        in_specs=[
            pl.BlockSpec(memory_space=pltpu.SMEM),
            pl.BlockSpec(memory_space=pltpu.VMEM),
            pl.BlockSpec(memory_space=pltpu.VMEM),
        ],
        out_specs=(pl.BlockSpec(memory_space=pltpu.VMEM),
                   pl.BlockSpec(memory_space=pltpu.VMEM)),
    )(b, l0, l1)


def kernel(x, table, W, b):
    B, T = x.shape
    _, E = table.shape
    u = x.reshape(-1).astype(jnp.int32)
    h = u & 4095
    x_flat = u - h + ((h & 2047) << 1) + (h >> 11)
    w0 = W[:, 0].astype(jnp.float32)
    w1 = W[:, 1].astype(jnp.float32)
    table_rm = _transpose_table_tc(table.T)
    table_rm = table_rm.reshape(table_rm.shape[0] * 2, E)
    sc = _build_sc_logits(B, T, E, table_rm.shape[0], 2, 16, 4)
    l0, l1 = sc(x_flat, table_rm, w0, w1)
    o0, o1 = _softmax_tc(l0.reshape(128, -1), l1.reshape(128, -1), b)
    return jnp.stack([o0.reshape(B), o1.reshape(B)], axis=-1)


# speedup vs baseline: 3.5058x; 1.0884x over previous
import functools

import jax
import jax.numpy as jnp
from jax import lax
from jax.experimental import pallas as pl
from jax.experimental.pallas import tpu as pltpu
from jax.experimental.pallas import tpu_sc as plsc


def _build_sc_logits(B, T, E, V, NC, NS, R):
    NW = NC * NS
    RPW = B // NW
    NG = RPW // R
    TPG = R * T
    D = T * E
    EC = E // 16
    THA = ((T // 2 + 7) // 8) * 8
    THB = T - THA

    mesh = plsc.VectorSubcoreMesh(core_axis_name="c", subcore_axis_name="s",
                                  num_cores=NC, num_subcores=NS)

    @functools.partial(
        pl.kernel,
        out_type=(jax.ShapeDtypeStruct((B,), jnp.float32),
                  jax.ShapeDtypeStruct((B,), jnp.float32)),
        mesh=mesh,
        compiler_params=pltpu.CompilerParams(needs_layout_passes=False,
                                             use_tc_tiling_on_sc=False),
        scratch_types=[
            pltpu.VMEM((D,), jnp.float32),
            pltpu.VMEM((D,), jnp.float32),
            pltpu.VMEM((TPG,), jnp.int32),
            pltpu.VMEM((TPG,), jnp.int32),
            pltpu.VMEM((R * THA, E), jnp.float32),
            pltpu.VMEM((R * THB, E), jnp.float32),
            pltpu.VMEM((RPW * 16,), jnp.float32),
            pltpu.VMEM((RPW * 16,), jnp.float32),
            pltpu.VMEM((RPW,), jnp.float32),
            pltpu.VMEM((RPW,), jnp.float32),
            pltpu.SemaphoreType.DMA,
            pltpu.SemaphoreType.DMA,
            pltpu.SemaphoreType.DMA,
        ],
    )
    def sc_logits(x_hbm, tab1d_hbm, w0_hbm, w1_hbm, l0_hbm, l1_hbm,
                  w0v, w1v, idx0, idx1, bufA, bufB, p0, p1, o0v, o1v,
                  semA, semB, semI):
        tab_hbm = tab1d_hbm
        wid = lax.axis_index("s") * NC + lax.axis_index("c")
        tok_base = wid * (RPW * T)
        pltpu.sync_copy(w0_hbm, w0v)
        pltpu.sync_copy(w1_hbm, w1v)

        def issue_half(idxv, buf, sem, tok_off, ntok):
            for r in range(R):
                pltpu.async_copy(
                    tab_hbm.at[idxv.at[pl.ds(r * T + tok_off, ntok)]],
                    buf.at[pl.ds(r * ntok, ntok)], sem)

        def wait_half(buf, sem, ntok):
            for r in range(R):
                pltpu.make_async_copy(
                    tab_hbm.at[idx0.at[pl.ds(0, ntok)]],
                    buf.at[pl.ds(r * ntok, ntok)], sem).wait()

        def compute_half(buf, tok_off, ntok, accs):
            def body(i, accs):
                t = i // EC
                e0 = (i % EC) * 16
                w0c = w0v[pl.ds(tok_off * E + i * 16, 16)]
                w1c = w1v[pl.ds(tok_off * E + i * 16, 16)]
                out = []
                for r in range(R):
                    v = buf[r * ntok + t, pl.ds(e0, 16)]
                    out.append(accs[2 * r] + v * w0c)
                    out.append(accs[2 * r + 1] + v * w1c)
                return tuple(out)
            return lax.fori_loop(0, ntok * EC, body, accs, unroll=2)

        pltpu.sync_copy(x_hbm.at[pl.ds(tok_base, TPG)], idx0)
        issue_half(idx0, bufA, semA, 0, THA)
        issue_half(idx0, bufB, semB, THA, THB)
        pltpu.async_copy(x_hbm.at[pl.ds(tok_base + TPG, TPG)], idx1, semI)

        zero = jnp.zeros((16,), jnp.float32)

        @pl.loop(0, NG, step=2)
        def _groups(G):
            for p, (cur, nxt) in ((0, (idx0, idx1)), (1, (idx1, idx0))):
                g = G + p
                wait_half(bufA, semA, THA)
                accs = compute_half(bufA, 0, THA, (zero,) * (2 * R))

                @pl.when(g + 1 < NG)
                def _():
                    pltpu.make_async_copy(
                        x_hbm.at[pl.ds(tok_base, TPG)], nxt, semI).wait()
                    issue_half(nxt, bufA, semA, 0, THA)

                wait_half(bufB, semB, THB)
                accs = compute_half(bufB, THA, THB, accs)

                for r in range(R):
                    p0[pl.ds((g * R + r) * 16, 16)] = accs[2 * r]
                    p1[pl.ds((g * R + r) * 16, 16)] = accs[2 * r + 1]

                @pl.when(g + 1 < NG)
                def _():
                    issue_half(nxt, bufB, semB, THA, THB)

                @pl.when(g + 2 < NG)
                def _():
                    pltpu.async_copy(
                        x_hbm.at[pl.ds(tok_base + (g + 2) * TPG, TPG)],
                        cur, semI)

        lanes16 = lax.iota(jnp.int32, 16) * 16

        @pl.loop(0, RPW // 16)
        def _reduce(j):
            base = j * 256 + lanes16
            s0 = jnp.zeros((16,), jnp.float32)
            s1 = jnp.zeros((16,), jnp.float32)
            for c in range(16):
                s0 = s0 + plsc.load_gather(p0, [base + c])
                s1 = s1 + plsc.load_gather(p1, [base + c])
            o0v[pl.ds(j * 16, 16)] = s0
            o1v[pl.ds(j * 16, 16)] = s1

        out_base = wid * RPW
        pltpu.sync_copy(o0v, l0_hbm.at[pl.ds(out_base, RPW)])
        pltpu.sync_copy(o1v, l1_hbm.at[pl.ds(out_base, RPW)])

    return sc_logits


def _transpose_table_tc(tabT, C=4096):
    E, V = tabT.shape
    NB = (V + 2 * C - 1) // (2 * C)
    last_blk = (V + C - 1) // C - 1

    def body(l_ref, r_ref, o_ref):
        o_ref[...] = jnp.concatenate(
            [jnp.transpose(l_ref[...]), jnp.transpose(r_ref[...])], axis=1)

    return pl.pallas_call(
        body,
        grid=(NB,),
        in_specs=[
            pl.BlockSpec((E, C), lambda g: (0, jnp.minimum(2 * g, last_blk))),
            pl.BlockSpec((E, C),
                         lambda g: (0, jnp.minimum(2 * g + 1, last_blk))),
        ],
        out_specs=pl.BlockSpec((C, 2 * E), lambda g: (g, 0)),
        out_shape=jax.ShapeDtypeStruct((NB * C, 2 * E), jnp.float32),
    )(tabT, tabT)


def _softmax_tc(l0, l1, b):
    def body(b_ref, l0_ref, l1_ref, o0_ref, o1_ref):
        a0 = l0_ref[...] + b_ref[0]
        a1 = l1_ref[...] + b_ref[1]
        m = jnp.maximum(a0, a1)
        s = m + jnp.log(jnp.exp(a0 - m) + jnp.exp(a1 - m))
        o0_ref[...] = a0 - s
        o1_ref[...] = a1 - s

    return pl.pallas_call(
        body,
        out_shape=(jax.ShapeDtypeStruct(l0.shape, jnp.float32),
                   jax.ShapeDtypeStruct(l1.shape, jnp.float32)),
        in_specs=[
            pl.BlockSpec(memory_space=pltpu.SMEM),
            pl.BlockSpec(memory_space=pltpu.VMEM),
            pl.BlockSpec(memory_space=pltpu.VMEM),
        ],
        out_specs=(pl.BlockSpec(memory_space=pltpu.VMEM),
                   pl.BlockSpec(memory_space=pltpu.VMEM)),
    )(b, l0, l1)


def kernel(x, table, W, b):
    B, T = x.shape
    _, E = table.shape
    u = x.reshape(-1).astype(jnp.int32)
    h = u & 8191
    x_flat = u - h + ((h & 4095) << 1) + (h >> 12)
    w0 = W[:, 0].astype(jnp.float32)
    w1 = W[:, 1].astype(jnp.float32)
    table_rm = _transpose_table_tc(table.T)
    table_rm = table_rm.reshape(table_rm.shape[0] * 2, E)
    sc = _build_sc_logits(B, T, E, table_rm.shape[0], 2, 16, 4)
    l0, l1 = sc(x_flat, table_rm, w0, w1)
    o0, o1 = _softmax_tc(l0.reshape(128, -1), l1.reshape(128, -1), b)
    return jnp.stack([o0.reshape(B), o1.reshape(B)], axis=-1)


# speedup vs baseline: 3.6515x; 1.0416x over previous
import functools

import jax
import jax.numpy as jnp
from jax import lax
from jax.experimental import pallas as pl
from jax.experimental.pallas import tpu as pltpu
from jax.experimental.pallas import tpu_sc as plsc


def _build_sc_logits(B, T, E, V, NC, NS, R):
    NW = NC * NS
    RPW = B // NW
    NG = RPW // R
    TPG = R * T
    D = T * E
    EC = E // 16
    THA = ((T // 2 + 7) // 8) * 8
    THB = T - THA

    mesh = plsc.VectorSubcoreMesh(core_axis_name="c", subcore_axis_name="s",
                                  num_cores=NC, num_subcores=NS)

    @functools.partial(
        pl.kernel,
        out_type=(jax.ShapeDtypeStruct((B,), jnp.float32),
                  jax.ShapeDtypeStruct((B,), jnp.float32)),
        mesh=mesh,
        compiler_params=pltpu.CompilerParams(needs_layout_passes=False,
                                             use_tc_tiling_on_sc=False),
        scratch_types=[
            pltpu.VMEM((D,), jnp.float32),
            pltpu.VMEM((D,), jnp.float32),
            pltpu.VMEM((TPG,), jnp.int32),
            pltpu.VMEM((TPG,), jnp.int32),
            pltpu.VMEM((R * THA, E), jnp.float32),
            pltpu.VMEM((R * THB, E), jnp.float32),
            pltpu.VMEM((RPW * 16,), jnp.float32),
            pltpu.VMEM((RPW * 16,), jnp.float32),
            pltpu.VMEM((RPW,), jnp.float32),
            pltpu.VMEM((RPW,), jnp.float32),
            pltpu.SemaphoreType.DMA,
            pltpu.SemaphoreType.DMA,
            pltpu.SemaphoreType.DMA,
        ],
    )
    def sc_logits(x_hbm, tab1d_hbm, w0_hbm, w1_hbm, l0_hbm, l1_hbm,
                  w0v, w1v, idx0, idx1, bufA, bufB, p0, p1, o0v, o1v,
                  semA, semB, semI):
        tab_hbm = tab1d_hbm
        wid = lax.axis_index("s") * NC + lax.axis_index("c")
        tok_base = wid * (RPW * T)
        pltpu.sync_copy(w0_hbm, w0v)
        pltpu.sync_copy(w1_hbm, w1v)

        def issue_half(idxv, buf, sem, tok_off, ntok):
            for r in range(R):
                pltpu.async_copy(
                    tab_hbm.at[idxv.at[pl.ds(r * T + tok_off, ntok)]],
                    buf.at[pl.ds(r * ntok, ntok)], sem)

        def wait_half(buf, sem, ntok):
            for r in range(R):
                pltpu.make_async_copy(
                    tab_hbm.at[idx0.at[pl.ds(0, ntok)]],
                    buf.at[pl.ds(r * ntok, ntok)], sem).wait()

        def compute_half(buf, tok_off, ntok, accs):
            def body(i, accs):
                t = i // EC
                e0 = (i % EC) * 16
                w0c = w0v[pl.ds(tok_off * E + i * 16, 16)]
                w1c = w1v[pl.ds(tok_off * E + i * 16, 16)]
                out = []
                for r in range(R):
                    v = buf[r * ntok + t, pl.ds(e0, 16)]
                    out.append(accs[2 * r] + v * w0c)
                    out.append(accs[2 * r + 1] + v * w1c)
                return tuple(out)
            return lax.fori_loop(0, ntok * EC, body, accs, unroll=2)

        pltpu.sync_copy(x_hbm.at[pl.ds(tok_base, TPG)], idx0)
        issue_half(idx0, bufA, semA, 0, THA)
        issue_half(idx0, bufB, semB, THA, THB)
        pltpu.async_copy(x_hbm.at[pl.ds(tok_base + TPG, TPG)], idx1, semI)

        zero = jnp.zeros((16,), jnp.float32)

        @pl.loop(0, NG, step=2)
        def _groups(G):
            for p, (cur, nxt) in ((0, (idx0, idx1)), (1, (idx1, idx0))):
                g = G + p
                wait_half(bufA, semA, THA)
                accs = compute_half(bufA, 0, THA, (zero,) * (2 * R))

                @pl.when(g + 1 < NG)
                def _():
                    pltpu.make_async_copy(
                        x_hbm.at[pl.ds(tok_base, TPG)], nxt, semI).wait()
                    issue_half(nxt, bufA, semA, 0, THA)

                wait_half(bufB, semB, THB)
                accs = compute_half(bufB, THA, THB, accs)

                for r in range(R):
                    p0[pl.ds((g * R + r) * 16, 16)] = accs[2 * r]
                    p1[pl.ds((g * R + r) * 16, 16)] = accs[2 * r + 1]

                @pl.when(g + 1 < NG)
                def _():
                    issue_half(nxt, bufB, semB, THA, THB)

                @pl.when(g + 2 < NG)
                def _():
                    pltpu.async_copy(
                        x_hbm.at[pl.ds(tok_base + (g + 2) * TPG, TPG)],
                        cur, semI)

        lanes16 = lax.iota(jnp.int32, 16) * 16

        @pl.loop(0, RPW // 16)
        def _reduce(j):
            base = j * 256 + lanes16
            s0 = jnp.zeros((16,), jnp.float32)
            s1 = jnp.zeros((16,), jnp.float32)
            for c in range(16):
                s0 = s0 + plsc.load_gather(p0, [base + c])
                s1 = s1 + plsc.load_gather(p1, [base + c])
            o0v[pl.ds(j * 16, 16)] = s0
            o1v[pl.ds(j * 16, 16)] = s1

        out_base = wid * RPW
        pltpu.sync_copy(o0v, l0_hbm.at[pl.ds(out_base, RPW)])
        pltpu.sync_copy(o1v, l1_hbm.at[pl.ds(out_base, RPW)])

    return sc_logits


def _transpose_table_tc(tabT, C=8192):
    E, V = tabT.shape
    NB = (V + 2 * C - 1) // (2 * C)
    last_blk = (V + C - 1) // C - 1

    def body(l_ref, r_ref, o_ref):
        o_ref[...] = jnp.concatenate(
            [jnp.transpose(l_ref[...]), jnp.transpose(r_ref[...])], axis=1)

    return pl.pallas_call(
        body,
        grid=(NB,),
        in_specs=[
            pl.BlockSpec((E, C), lambda g: (0, jnp.minimum(2 * g, last_blk))),
            pl.BlockSpec((E, C),
                         lambda g: (0, jnp.minimum(2 * g + 1, last_blk))),
        ],
        out_specs=pl.BlockSpec((C, 2 * E), lambda g: (g, 0)),
        out_shape=jax.ShapeDtypeStruct((NB * C, 2 * E), jnp.float32),
    )(tabT, tabT)


def _softmax_tc(l0, l1, b):
    def body(b_ref, l0_ref, l1_ref, o0_ref, o1_ref):
        a0 = l0_ref[...] + b_ref[0]
        a1 = l1_ref[...] + b_ref[1]
        m = jnp.maximum(a0, a1)
        s = m + jnp.log(jnp.exp(a0 - m) + jnp.exp(a1 - m))
        o0_ref[...] = a0 - s
        o1_ref[...] = a1 - s

    return pl.pallas_call(
        body,
        out_shape=(jax.ShapeDtypeStruct(l0.shape, jnp.float32),
                   jax.ShapeDtypeStruct(l1.shape, jnp.float32)),
        in_specs=[
            pl.BlockSpec(memory_space=pltpu.SMEM),
            pl.BlockSpec(memory_space=pltpu.VMEM),
            pl.BlockSpec(memory_space=pltpu.VMEM),
        ],
        out_specs=(pl.BlockSpec(memory_space=pltpu.VMEM),
                   pl.BlockSpec(memory_space=pltpu.VMEM)),
    )(b, l0, l1)


def kernel(x, table, W, b):
    B, T = x.shape
    _, E = table.shape
    u = x.reshape(-1).astype(jnp.int32)
    h = u & 16383
    x_flat = u - h + ((h & 8191) << 1) + (h >> 13)
    w0 = W[:, 0].astype(jnp.float32)
    w1 = W[:, 1].astype(jnp.float32)
    table_rm = _transpose_table_tc(table.T)
    table_rm = table_rm.reshape(table_rm.shape[0] * 2, E)
    sc = _build_sc_logits(B, T, E, table_rm.shape[0], 2, 16, 4)
    l0, l1 = sc(x_flat, table_rm, w0, w1)
    o0, o1 = _softmax_tc(l0.reshape(128, -1), l1.reshape(128, -1), b)
    return jnp.stack([o0.reshape(B), o1.reshape(B)], axis=-1)


# speedup vs baseline: 3.7292x; 1.0213x over previous
import functools

import jax
import jax.numpy as jnp
from jax import lax
from jax.experimental import pallas as pl
from jax.experimental.pallas import tpu as pltpu
from jax.experimental.pallas import tpu_sc as plsc


def _build_sc_logits(B, T, E, V, NC, NS, R):
    NW = NC * NS
    RPW = B // NW
    NG = RPW // R
    TPG = R * T
    D = T * E
    EC = E // 16
    THA = ((T // 2 + 7) // 8) * 8
    THB = T - THA

    mesh = plsc.VectorSubcoreMesh(core_axis_name="c", subcore_axis_name="s",
                                  num_cores=NC, num_subcores=NS)

    @functools.partial(
        pl.kernel,
        out_type=(jax.ShapeDtypeStruct((B,), jnp.float32),
                  jax.ShapeDtypeStruct((B,), jnp.float32)),
        mesh=mesh,
        compiler_params=pltpu.CompilerParams(needs_layout_passes=False,
                                             use_tc_tiling_on_sc=False),
        scratch_types=[
            pltpu.VMEM((D,), jnp.float32),
            pltpu.VMEM((D,), jnp.float32),
            pltpu.VMEM((TPG,), jnp.int32),
            pltpu.VMEM((TPG,), jnp.int32),
            pltpu.VMEM((R * THA, E), jnp.float32),
            pltpu.VMEM((R * THB, E), jnp.float32),
            pltpu.VMEM((RPW * 16,), jnp.float32),
            pltpu.VMEM((RPW * 16,), jnp.float32),
            pltpu.VMEM((RPW,), jnp.float32),
            pltpu.VMEM((RPW,), jnp.float32),
            pltpu.SemaphoreType.DMA,
            pltpu.SemaphoreType.DMA,
            pltpu.SemaphoreType.DMA,
        ],
    )
    def sc_logits(x_hbm, tab1d_hbm, w0_hbm, w1_hbm, l0_hbm, l1_hbm,
                  w0v, w1v, idx0, idx1, bufA, bufB, p0, p1, o0v, o1v,
                  semA, semB, semI):
        tab_hbm = tab1d_hbm
        wid = lax.axis_index("s") * NC + lax.axis_index("c")
        tok_base = wid * (RPW * T)
        pltpu.sync_copy(w0_hbm, w0v)
        pltpu.sync_copy(w1_hbm, w1v)

        def issue_half(idxv, buf, sem, tok_off, ntok):
            for r in range(R):
                pltpu.async_copy(
                    tab_hbm.at[idxv.at[pl.ds(r * T + tok_off, ntok)]],
                    buf.at[pl.ds(r * ntok, ntok)], sem)

        def wait_half(buf, sem, ntok):
            for r in range(R):
                pltpu.make_async_copy(
                    tab_hbm.at[idx0.at[pl.ds(0, ntok)]],
                    buf.at[pl.ds(r * ntok, ntok)], sem).wait()

        def compute_half(buf, tok_off, ntok, accs):
            def body(i, accs):
                t = i // EC
                e0 = (i % EC) * 16
                w0c = w0v[pl.ds(tok_off * E + i * 16, 16)]
                w1c = w1v[pl.ds(tok_off * E + i * 16, 16)]
                out = []
                for r in range(R):
                    v = buf[r * ntok + t, pl.ds(e0, 16)]
                    out.append(accs[2 * r] + v * w0c)
                    out.append(accs[2 * r + 1] + v * w1c)
                return tuple(out)
            return lax.fori_loop(0, ntok * EC, body, accs, unroll=2)

        pltpu.sync_copy(x_hbm.at[pl.ds(tok_base, TPG)], idx0)
        issue_half(idx0, bufA, semA, 0, THA)
        issue_half(idx0, bufB, semB, THA, THB)
        pltpu.async_copy(x_hbm.at[pl.ds(tok_base + TPG, TPG)], idx1, semI)

        zero = jnp.zeros((16,), jnp.float32)

        @pl.loop(0, NG, step=2)
        def _groups(G):
            for p, (cur, nxt) in ((0, (idx0, idx1)), (1, (idx1, idx0))):
                g = G + p
                wait_half(bufA, semA, THA)
                accs = compute_half(bufA, 0, THA, (zero,) * (2 * R))

                @pl.when(g + 1 < NG)
                def _():
                    pltpu.make_async_copy(
                        x_hbm.at[pl.ds(tok_base, TPG)], nxt, semI).wait()
                    issue_half(nxt, bufA, semA, 0, THA)

                wait_half(bufB, semB, THB)
                accs = compute_half(bufB, THA, THB, accs)

                for r in range(R):
                    p0[pl.ds((g * R + r) * 16, 16)] = accs[2 * r]
                    p1[pl.ds((g * R + r) * 16, 16)] = accs[2 * r + 1]

                @pl.when(g + 1 < NG)
                def _():
                    issue_half(nxt, bufB, semB, THA, THB)

                @pl.when(g + 2 < NG)
                def _():
                    pltpu.async_copy(
                        x_hbm.at[pl.ds(tok_base + (g + 2) * TPG, TPG)],
                        cur, semI)

        lanes16 = lax.iota(jnp.int32, 16) * 16

        @pl.loop(0, RPW // 16)
        def _reduce(j):
            base = j * 256 + lanes16
            s0 = jnp.zeros((16,), jnp.float32)
            s1 = jnp.zeros((16,), jnp.float32)
            for c in range(16):
                s0 = s0 + plsc.load_gather(p0, [base + c])
                s1 = s1 + plsc.load_gather(p1, [base + c])
            o0v[pl.ds(j * 16, 16)] = s0
            o1v[pl.ds(j * 16, 16)] = s1

        out_base = wid * RPW
        pltpu.sync_copy(o0v, l0_hbm.at[pl.ds(out_base, RPW)])
        pltpu.sync_copy(o1v, l1_hbm.at[pl.ds(out_base, RPW)])

    return sc_logits


def _transpose_table_tc(tabT, C=16384):
    E, V = tabT.shape
    NB = (V + 2 * C - 1) // (2 * C)
    last_blk = (V + C - 1) // C - 1

    def body(l_ref, r_ref, o_ref):
        o_ref[...] = jnp.concatenate(
            [jnp.transpose(l_ref[...]), jnp.transpose(r_ref[...])], axis=1)

    return pl.pallas_call(
        body,
        grid=(NB,),
        in_specs=[
            pl.BlockSpec((E, C), lambda g: (0, jnp.minimum(2 * g, last_blk))),
            pl.BlockSpec((E, C),
                         lambda g: (0, jnp.minimum(2 * g + 1, last_blk))),
        ],
        out_specs=pl.BlockSpec((C, 2 * E), lambda g: (g, 0)),
        out_shape=jax.ShapeDtypeStruct((NB * C, 2 * E), jnp.float32),
    )(tabT, tabT)


def _softmax_tc(l0, l1, b):
    def body(b_ref, l0_ref, l1_ref, o0_ref, o1_ref):
        a0 = l0_ref[...] + b_ref[0]
        a1 = l1_ref[...] + b_ref[1]
        m = jnp.maximum(a0, a1)
        s = m + jnp.log(jnp.exp(a0 - m) + jnp.exp(a1 - m))
        o0_ref[...] = a0 - s
        o1_ref[...] = a1 - s

    return pl.pallas_call(
        body,
        out_shape=(jax.ShapeDtypeStruct(l0.shape, jnp.float32),
                   jax.ShapeDtypeStruct(l1.shape, jnp.float32)),
        in_specs=[
            pl.BlockSpec(memory_space=pltpu.SMEM),
            pl.BlockSpec(memory_space=pltpu.VMEM),
            pl.BlockSpec(memory_space=pltpu.VMEM),
        ],
        out_specs=(pl.BlockSpec(memory_space=pltpu.VMEM),
                   pl.BlockSpec(memory_space=pltpu.VMEM)),
    )(b, l0, l1)


def kernel(x, table, W, b):
    B, T = x.shape
    _, E = table.shape
    u = x.reshape(-1).astype(jnp.int32)
    h = u & 32767
    x_flat = u - h + ((h & 16383) << 1) + (h >> 14)
    w0 = W[:, 0].astype(jnp.float32)
    w1 = W[:, 1].astype(jnp.float32)
    table_rm = _transpose_table_tc(table.T)
    table_rm = table_rm.reshape(table_rm.shape[0] * 2, E)
    sc = _build_sc_logits(B, T, E, table_rm.shape[0], 2, 16, 4)
    l0, l1 = sc(x_flat, table_rm, w0, w1)
    o0, o1 = _softmax_tc(l0.reshape(128, -1), l1.reshape(128, -1), b)
    return jnp.stack([o0.reshape(B), o1.reshape(B)], axis=-1)
